# Initial kernel scaffold; baseline (speedup 1.0000x reference)
import jax
import jax.numpy as jnp
from jax.experimental import pallas as pl


def kernel(x, e, W):
    raise NotImplementedError("write your pallas kernel here")


# speedup vs baseline: 1.0510x; 1.0510x over previous
import functools

import jax
import jax.numpy as jnp
from jax import lax
from jax.experimental import pallas as pl
from jax.experimental.pallas import tpu as pltpu
from jax.experimental.pallas import tpu_sc as plsc

_N = 131072
_D = 64
_K = 512
_COMMIT = 0.25

_ROWS = 1024
_NW = 32
_CHUNK = 128


def _tc_body(at_ref, e_ref, wt_ref, enc_ref, loss_ref):
    i = pl.program_id(0)
    e = e_ref[...]
    wt = wt_ref[...]
    mm = jnp.dot(e, wt, preferred_element_type=jnp.float32)
    se = jnp.sum(e * e, axis=1, keepdims=True)
    sw = jnp.sum(wt * wt, axis=0, keepdims=True)
    d = (se + sw) - 2.0 * mm

    at = at_ref[...]
    lo = jnp.where(at == 5.0, 0,
         jnp.where(at == 6.0, 378,
         jnp.where(at == 7.0, 434, 489)))
    hi = jnp.where(at == 5.0, 377,
         jnp.where(at == 6.0, 433,
         jnp.where(at == 7.0, 488, 511)))
    col = lax.broadcasted_iota(jnp.int32, d.shape, 1)
    dm = jnp.where((col >= lo) & (col < hi), d, jnp.inf)

    enc_ref[...] = jnp.argmin(dm, axis=1).astype(jnp.int32)[:, None]
    part = jnp.sum(jnp.min(dm, axis=1)).reshape(1, 1)

    @pl.when(i == 0)
    def _():
        loss_ref[...] = jnp.zeros((1, 1), jnp.float32)

    loss_ref[...] += part


def _tc_encode(at, e, wt):
    n = e.shape[0]
    grid = n // _ROWS
    return pl.pallas_call(
        _tc_body,
        grid=(grid,),
        in_specs=[
            pl.BlockSpec((_ROWS, 1), lambda i: (i, 0)),
            pl.BlockSpec((_ROWS, _D), lambda i: (i, 0)),
            pl.BlockSpec((_D, _K), lambda i: (0, 0)),
        ],
        out_specs=[
            pl.BlockSpec((_ROWS, 1), lambda i: (i, 0)),
            pl.BlockSpec((1, 1), lambda i: (0, 0)),
        ],
        out_shape=[
            jax.ShapeDtypeStruct((n, 1), jnp.int32),
            jax.ShapeDtypeStruct((1, 1), jnp.float32),
        ],
    )(at, e, wt)


def _sc_gather(table, idx):
    n = idx.shape[0]
    rows_per_w = n // _NW
    nchunk = rows_per_w // _CHUNK
    mesh = plsc.VectorSubcoreMesh(core_axis_name="c", subcore_axis_name="s")

    @functools.partial(
        pl.kernel,
        mesh=mesh,
        compiler_params=pltpu.CompilerParams(use_tc_tiling_on_sc=False),
        out_type=jax.ShapeDtypeStruct((n, _D), jnp.float32),
        scratch_types=[
            pltpu.VMEM((_CHUNK,), jnp.int32),
            pltpu.VMEM((_CHUNK, _D), jnp.float32),
            pltpu.SemaphoreType.DMA,
        ],
    )
    def k(table_hbm, idx_hbm, out_hbm, idx_v, rows_v, sem):
        wid = lax.axis_index("s") * 2 + lax.axis_index("c")
        base = wid * rows_per_w

        def body(j, carry):
            row0 = pl.multiple_of(base + j * _CHUNK, _CHUNK)
            pltpu.sync_copy(idx_hbm.at[pl.ds(row0, _CHUNK)], idx_v)
            pltpu.async_copy(table_hbm.at[idx_v], rows_v, sem).wait()
            pltpu.sync_copy(rows_v, out_hbm.at[pl.ds(row0, _CHUNK)])
            return carry

        lax.fori_loop(0, nchunk, body, 0)

    return k(table, idx)


def kernel(x, e, W):
    at = x[:, 0:1]
    wt = W.T
    enc2d, loss_sum = _tc_encode(at, e, wt)
    enc = enc2d.reshape(-1)
    quantized = _sc_gather(W, enc)
    loss = loss_sum[0, 0] * ((1.0 + _COMMIT) / (e.size))
    return quantized, loss


# speedup vs baseline: 1.0623x; 1.0108x over previous
import functools

import jax
import jax.numpy as jnp
from jax import lax
from jax.experimental import pallas as pl
from jax.experimental.pallas import tpu as pltpu
from jax.experimental.pallas import tpu_sc as plsc

_N = 131072
_D = 64
_K = 512
_COMMIT = 0.25

_ROWS = 1024
_NW = 32
_CHUNK = 128
_MACRO = 512
_GPC = _MACRO // _CHUNK


def _tc_body(at_ref, e_ref, wt_ref, enc_ref, loss_ref):
    i = pl.program_id(0)
    e = e_ref[...]
    wt = wt_ref[...]
    mm = jnp.dot(e, wt, preferred_element_type=jnp.float32)
    se = jnp.sum(e * e, axis=1, keepdims=True)
    sw = jnp.sum(wt * wt, axis=0, keepdims=True)
    d = (se + sw) - 2.0 * mm

    at = at_ref[...]
    lo = jnp.where(at == 5.0, 0,
         jnp.where(at == 6.0, 378,
         jnp.where(at == 7.0, 434, 489)))
    hi = jnp.where(at == 5.0, 377,
         jnp.where(at == 6.0, 433,
         jnp.where(at == 7.0, 488, 511)))
    col = lax.broadcasted_iota(jnp.int32, d.shape, 1)
    dm = jnp.where((col >= lo) & (col < hi), d, jnp.inf)

    enc_ref[...] = jnp.argmin(dm, axis=1).astype(jnp.int32)[:, None]
    part = jnp.sum(jnp.min(dm, axis=1)).reshape(1, 1)

    @pl.when(i == 0)
    def _():
        loss_ref[...] = jnp.zeros((1, 1), jnp.float32)

    loss_ref[...] += part


def _tc_encode(at, e, wt):
    n = e.shape[0]
    grid = n // _ROWS
    return pl.pallas_call(
        _tc_body,
        grid=(grid,),
        in_specs=[
            pl.BlockSpec((_ROWS, 1), lambda i: (i, 0)),
            pl.BlockSpec((_ROWS, _D), lambda i: (i, 0)),
            pl.BlockSpec((_D, _K), lambda i: (0, 0)),
        ],
        out_specs=[
            pl.BlockSpec((_ROWS, 1), lambda i: (i, 0)),
            pl.BlockSpec((1, 1), lambda i: (0, 0)),
        ],
        out_shape=[
            jax.ShapeDtypeStruct((n, 1), jnp.int32),
            jax.ShapeDtypeStruct((1, 1), jnp.float32),
        ],
    )(at, e, wt)


def _sc_gather(table, idx):
    n = idx.shape[0]
    rows_per_w = n // _NW
    nmacro = rows_per_w // _MACRO
    mesh = plsc.VectorSubcoreMesh(core_axis_name="c", subcore_axis_name="s")

    @functools.partial(
        pl.kernel,
        mesh=mesh,
        compiler_params=pltpu.CompilerParams(use_tc_tiling_on_sc=False),
        out_type=jax.ShapeDtypeStruct((n, _D), jnp.float32),
        scratch_types=[
            pltpu.VMEM((rows_per_w,), jnp.int32),
            pltpu.VMEM((_MACRO, _D), jnp.float32),
            pltpu.VMEM((_MACRO, _D), jnp.float32),
            pltpu.SemaphoreType.DMA,
            pltpu.SemaphoreType.DMA,
            pltpu.SemaphoreType.DMA,
            pltpu.SemaphoreType.DMA,
        ],
    )
    def k(table_hbm, idx_hbm, out_hbm, idx_v, rows0, rows1, sg0, sg1, st0, st1):
        wid = lax.axis_index("s") * 2 + lax.axis_index("c")
        base = pl.multiple_of(wid * rows_per_w, rows_per_w)
        pltpu.sync_copy(idx_hbm.at[pl.ds(base, rows_per_w)], idx_v)

        def fire(m, rows, sem):
            descs = []
            for c in range(_GPC):
                off = pl.multiple_of(m * _MACRO + c * _CHUNK, _CHUNK)
                descs.append(pltpu.async_copy(
                    table_hbm.at[idx_v.at[pl.ds(off, _CHUNK)]],
                    rows.at[pl.ds(c * _CHUNK, _CHUNK)],
                    sem))
            return descs

        def store(m, rows, sem):
            row0 = pl.multiple_of(base + m * _MACRO, _MACRO)
            return pltpu.async_copy(rows, out_hbm.at[pl.ds(row0, _MACRO)], sem)

        def body(g, carry):
            a = g * 2
            b = a + 1

            @pl.when(g > 0)
            def _():
                pltpu.make_async_copy(
                    rows0, out_hbm.at[pl.ds(base, _MACRO)], st0).wait()
                pltpu.make_async_copy(
                    rows1, out_hbm.at[pl.ds(base, _MACRO)], st1).wait()

            ga = fire(a, rows0, sg0)
            gb = fire(b, rows1, sg1)
            for dsc in ga:
                dsc.wait()
            store(a, rows0, st0)
            for dsc in gb:
                dsc.wait()
            store(b, rows1, st1)
            return carry

        lax.fori_loop(0, nmacro // 2, body, 0)
        pltpu.make_async_copy(rows0, out_hbm.at[pl.ds(base, _MACRO)], st0).wait()
        pltpu.make_async_copy(rows1, out_hbm.at[pl.ds(base, _MACRO)], st1).wait()

    return k(table, idx)


def kernel(x, e, W):
    at = x[:, 0:1]
    wt = W.T
    enc2d, loss_sum = _tc_encode(at, e, wt)
    enc = enc2d.reshape(-1)
    quantized = _sc_gather(W, enc)
    loss = loss_sum[0, 0] * ((1.0 + _COMMIT) / (e.size))
    return quantized, loss
